# Initial kernel scaffold; baseline (speedup 1.0000x reference)
import functools

import jax
import jax.numpy as jnp
from jax import lax
from jax.experimental import pallas as pl
from jax.experimental.pallas import tpu as pltpu
from jax.experimental.pallas import tpu_sc as plsc

B = 16384
D = 32
NC = 2
NS = 16
NW = NC * NS
BPW = B // NW
C = 128
NCH = BPW // C

NJ = (1, 1, 20, 4, 8)
ROW0 = (0, 4, 8, 88, 104)
ROWS = 136

H1, H2 = 64, 32


def _pack_indices(user, movie, actor, country, movie_type):
  cols = []
  for idx2d in (user[:, None], movie[:, None], actor, country, movie_type):
    nj = idx2d.shape[1]
    x = idx2d.astype(jnp.int32).reshape(NW, NCH, C, nj)
    cols.append(x.transpose(0, 3, 1, 2).reshape(NW, nj * NCH, C))
  return jnp.concatenate(cols, axis=1)


_mesh = plsc.VectorSubcoreMesh(core_axis_name="c", subcore_axis_name="s")


@functools.partial(
    pl.kernel,
    out_type=[jax.ShapeDtypeStruct((B, D), jnp.float32) for _ in range(5)],
    mesh=_mesh,
    scratch_types=[
        pltpu.VMEM((ROWS, C), jnp.int32),
        pltpu.VMEM((BPW, D), jnp.float32),
        pltpu.VMEM((BPW, D), jnp.float32),
        pltpu.VMEM((BPW, D), jnp.float32),
        pltpu.VMEM((BPW, D), jnp.float32),
        pltpu.VMEM((BPW, D), jnp.float32),
        pltpu.SemaphoreType.DMA,
        pltpu.SemaphoreType.DMA,
    ],
)
def _sc_gather(u_tab, m_tab, a_tab, c_tab, t_tab, idx_hbm,
               uo, mo, ao, co, to,
               idx_v, ua, ma, aa, ca, ta, sem0, sema):
  wid = lax.axis_index("s") * NC + lax.axis_index("c")
  base = wid * BPW
  pltpu.sync_copy(idx_hbm.at[wid], idx_v)

  wave0 = []
  for acc, tab, r0 in ((ua, u_tab, ROW0[0]), (ma, m_tab, ROW0[1]),
                       (aa, a_tab, ROW0[2]), (ca, c_tab, ROW0[3]),
                       (ta, t_tab, ROW0[4])):
    for c in range(NCH):
      wave0.append(pltpu.async_copy(tab.at[idx_v.at[r0 + c]],
                                    acc.at[pl.ds(c * C, C)], sem0))
  for d in wave0:
    d.wait()

  def add_slots(tab, acc, r0, nj):
    def body(j, carry):
      ds = [pltpu.async_copy(tab.at[idx_v.at[r0 + j * NCH + c]],
                             acc.at[pl.ds(c * C, C)], sema, add=True)
            for c in range(NCH)]
      for d in ds:
        d.wait()
      return carry
    lax.fori_loop(1, nj, body, 0)

  add_slots(a_tab, aa, ROW0[2], NJ[2])
  add_slots(c_tab, ca, ROW0[3], NJ[3])
  add_slots(t_tab, ta, ROW0[4], NJ[4])

  for acc, out in ((ua, uo), (ma, mo), (aa, ao), (ca, co), (ta, to)):
    pltpu.sync_copy(acc, out.at[pl.ds(base, BPW)])


BT = 2048


def _mlp_body(u, m, a, c, t, w1, b1, w2, b2, w3t, b3, o):
  h = jnp.dot(u[...], w1[0:D, :], preferred_element_type=jnp.float32)
  h += jnp.dot(m[...], w1[D:2 * D, :], preferred_element_type=jnp.float32)
  h += jnp.dot(a[...], w1[2 * D:3 * D, :], preferred_element_type=jnp.float32)
  h += jnp.dot(c[...], w1[3 * D:4 * D, :], preferred_element_type=jnp.float32)
  h += jnp.dot(t[...], w1[4 * D:5 * D, :], preferred_element_type=jnp.float32)
  h = jnp.maximum(h + b1[...], 0.0)
  h = jnp.maximum(jnp.dot(h, w2[...], preferred_element_type=jnp.float32)
                  + b2[...], 0.0)
  o[...] = jnp.sum(h * w3t[...], axis=1) + b3[0, 0]


def _mlp(ue, me, ae, ce, te, w1s, b1, w2, b2, w3t, b3):
  emb_spec = pl.BlockSpec((BT, D), lambda i: (i, 0))
  full = lambda *s: pl.BlockSpec(s, lambda i: tuple(0 for _ in s))
  return pl.pallas_call(
      _mlp_body,
      grid=(B // BT,),
      in_specs=[emb_spec] * 5 + [full(5 * D, H1), full(1, H1), full(H1, H2),
                                 full(1, H2), full(1, H2), full(1, 1)],
      out_specs=pl.BlockSpec((BT,), lambda i: (i,)),
      out_shape=jax.ShapeDtypeStruct((B,), jnp.float32),
  )(ue, me, ae, ce, te, w1s, b1, w2, b2, w3t, b3)


def kernel(user, movie, actor, country, movie_type,
           user_table, movie_table, actor_table, country_table, type_table,
           W1, b1, W2, b2, W3, b3):
  packed = _pack_indices(user, movie, actor, country, movie_type)
  ue, me, ae, ce, te = _sc_gather(user_table, movie_table, actor_table,
                                  country_table, type_table, packed)
  scale = jnp.concatenate([
      jnp.ones((2 * D,), jnp.float32),
      jnp.full((D,), 1.0 / NJ[2], jnp.float32),
      jnp.full((D,), 1.0 / NJ[3], jnp.float32),
      jnp.full((D,), 1.0 / NJ[4], jnp.float32),
  ])[:, None]
  w1s = W1 * scale
  return _mlp(ue, me, ae, ce, te, w1s, b1.reshape(1, H1), W2,
              b2.reshape(1, H2), W3.reshape(1, H2), b3.reshape(1, 1))


# speedup vs baseline: 4.4246x; 4.4246x over previous
import functools

import jax
import jax.numpy as jnp
from jax import lax
from jax.experimental import pallas as pl
from jax.experimental.pallas import tpu as pltpu
from jax.experimental.pallas import tpu_sc as plsc

B = 16384
D = 32
NC = 2
NS = 16
NW = NC * NS
BPW = B // NW
C = 128
NCH = BPW // C

NJ = (1, 1, 20, 4, 8)
ROW0 = (0, 4, 8, 88, 104)
ROWS = 136

H1, H2 = 64, 32


def _pack_indices(user, movie, actor, country, movie_type):
  cols = []
  for idx2d in (user[:, None], movie[:, None], actor, country, movie_type):
    nj = idx2d.shape[1]
    x = idx2d.astype(jnp.int32).reshape(NW, NCH, C, nj)
    cols.append(x.transpose(0, 3, 1, 2).reshape(NW, nj * NCH, C))
  return jnp.concatenate(cols, axis=1)


def _sc_gather_body(u_tab, m_tab, a_tab, c_tab, t_tab, idx_hbm,
                    uo, mo, ao, co, to,
                    idx_v, ua, ma, aa, ca, ta, sem0, sema):
  wid = lax.axis_index("s") * NC + lax.axis_index("c")
  base = wid * BPW
  pltpu.sync_copy(idx_hbm.at[wid], idx_v)

  wave0 = []
  for acc, tab, r0 in ((ua, u_tab, ROW0[0]), (ma, m_tab, ROW0[1]),
                       (aa, a_tab, ROW0[2]), (ca, c_tab, ROW0[3]),
                       (ta, t_tab, ROW0[4])):
    for c in range(NCH):
      wave0.append(pltpu.async_copy(tab.at[idx_v.at[r0 + c]],
                                    acc.at[pl.ds(c * C, C)], sem0))
  for d in wave0:
    d.wait()

  def add_slots(tab, acc, r0, nj):
    def body(j, carry):
      ds = [pltpu.async_copy(tab.at[idx_v.at[r0 + j * NCH + c]],
                             acc.at[pl.ds(c * C, C)], sema, add=True)
            for c in range(NCH)]
      for d in ds:
        d.wait()
      return carry
    lax.fori_loop(1, nj, body, 0)

  add_slots(a_tab, aa, ROW0[2], NJ[2])
  add_slots(c_tab, ca, ROW0[3], NJ[3])
  add_slots(t_tab, ta, ROW0[4], NJ[4])

  for acc, out in ((ua, uo), (ma, mo), (aa, ao), (ca, co), (ta, to)):
    pltpu.sync_copy(acc, out.at[pl.ds(base, BPW)])


@functools.cache
def _sc_gather():
  mesh = plsc.VectorSubcoreMesh(core_axis_name="c", subcore_axis_name="s",
                                num_cores=NC, num_subcores=NS)
  return pl.kernel(
      _sc_gather_body,
      out_type=[jax.ShapeDtypeStruct((B, D), jnp.float32) for _ in range(5)],
      mesh=mesh,
      compiler_params=pltpu.CompilerParams(use_tc_tiling_on_sc=False),
      scratch_types=[
          pltpu.VMEM((ROWS, C), jnp.int32),
          pltpu.VMEM((BPW, D), jnp.float32),
          pltpu.VMEM((BPW, D), jnp.float32),
          pltpu.VMEM((BPW, D), jnp.float32),
          pltpu.VMEM((BPW, D), jnp.float32),
          pltpu.VMEM((BPW, D), jnp.float32),
          pltpu.SemaphoreType.DMA,
          pltpu.SemaphoreType.DMA,
      ],
  )


BT = 2048


def _mlp_body(u, m, a, c, t, w1, b1, w2, b2, w3t, b3, o):
  h = jnp.dot(u[...], w1[0:D, :], preferred_element_type=jnp.float32)
  h += jnp.dot(m[...], w1[D:2 * D, :], preferred_element_type=jnp.float32)
  h += jnp.dot(a[...], w1[2 * D:3 * D, :], preferred_element_type=jnp.float32)
  h += jnp.dot(c[...], w1[3 * D:4 * D, :], preferred_element_type=jnp.float32)
  h += jnp.dot(t[...], w1[4 * D:5 * D, :], preferred_element_type=jnp.float32)
  h = jnp.maximum(h + b1[...], 0.0)
  h = jnp.maximum(jnp.dot(h, w2[...], preferred_element_type=jnp.float32)
                  + b2[...], 0.0)
  o[...] = jnp.sum(h * w3t[...], axis=1) + b3[0, 0]


def _mlp(ue, me, ae, ce, te, w1s, b1, w2, b2, w3t, b3):
  emb_spec = pl.BlockSpec((BT, D), lambda i: (i, 0))
  full = lambda *s: pl.BlockSpec(s, lambda i: tuple(0 for _ in s))
  return pl.pallas_call(
      _mlp_body,
      grid=(B // BT,),
      in_specs=[emb_spec] * 5 + [full(5 * D, H1), full(1, H1), full(H1, H2),
                                 full(1, H2), full(1, H2), full(1, 1)],
      out_specs=pl.BlockSpec((BT,), lambda i: (i,)),
      out_shape=jax.ShapeDtypeStruct((B,), jnp.float32),
  )(ue, me, ae, ce, te, w1s, b1, w2, b2, w3t, b3)


def kernel(user, movie, actor, country, movie_type,
           user_table, movie_table, actor_table, country_table, type_table,
           W1, b1, W2, b2, W3, b3):
  packed = _pack_indices(user, movie, actor, country, movie_type)
  ue, me, ae, ce, te = _sc_gather()(user_table, movie_table, actor_table,
                                    country_table, type_table, packed)
  scale = jnp.concatenate([
      jnp.ones((2 * D,), jnp.float32),
      jnp.full((D,), 1.0 / NJ[2], jnp.float32),
      jnp.full((D,), 1.0 / NJ[3], jnp.float32),
      jnp.full((D,), 1.0 / NJ[4], jnp.float32),
  ])[:, None]
  w1s = W1 * scale
  return _mlp(ue, me, ae, ce, te, w1s, b1.reshape(1, H1), W2,
              b2.reshape(1, H2), W3.reshape(1, H2), b3.reshape(1, 1))
